# baseline (speedup 1.0000x reference)
import functools

import jax
import jax.numpy as jnp
from jax import lax
from jax.experimental import pallas as pl
from jax.experimental.pallas import tpu as pltpu
from jax.experimental.pallas import tpu_sc as plsc

D = 128
B = 16384

_info = plsc.get_sparse_core_info()
NC, NS, L = _info.num_cores, _info.num_subcores, _info.num_lanes
NW = NC * NS
E_PER_W = B // NW
CHUNK = 128
N_CHUNKS = E_PER_W // CHUNK

_IB = lax.GatherScatterMode.PROMISE_IN_BOUNDS


_DNUMS = lax.GatherDimensionNumbers(
    offset_dims=(), collapsed_slice_dims=(0,), start_index_map=(0,))


def _perm(x, idx):
    return lax.gather(x, idx[:, None], _DNUMS, slice_sizes=(1,), mode=_IB)


def _sums_kernel(heads_hbm, tails_hbm, in_hbm, out_hbm, w_hbm, sums_hbm,
                 hv0, tv0, ub0, vb0, hv1, tv1, ub1, vb1, wb, ob,
                 su0, sv0, su1, sv1):
    wid = lax.axis_index("s") * NC + lax.axis_index("c")
    lane = lax.iota(jnp.int32, L)
    lane0 = lane == 0
    hi_idx = (lane & 7) + 8
    p4_idx = (lane & 3) + 4
    p2_idx = (lane & 1) + 2
    p1_idx = (lane & 0) + 1

    pltpu.sync_copy(w_hbm, wb)
    wv = [wb[pl.ds(16 * j, 16)] for j in range(8)]
    bufs = [(hv0, tv0, ub0, vb0, su0, sv0), (hv1, tv1, ub1, vb1, su1, sv1)]

    def start(ci):
        hv, tv, ub, vb, su, sv = bufs[ci & 1]
        base = wid * E_PER_W + ci * CHUNK
        pltpu.sync_copy(heads_hbm.at[pl.ds(base, CHUNK)], hv)
        pltpu.sync_copy(tails_hbm.at[pl.ds(base, CHUNK)], tv)
        return (pltpu.async_copy(in_hbm.at[hv], ub, su),
                pltpu.async_copy(out_hbm.at[tv], vb, sv))

    pend = start(0)
    for ci in range(N_CHUNKS):
        nxt = start(ci + 1) if ci + 1 < N_CHUNKS else None
        pend[0].wait()
        pend[1].wait()
        _, _, ub, vb, _, _ = bufs[ci & 1]

        @plsc.parallel_loop(0, CHUNK, unroll=4)
        def edge_body(e):
            acc = None
            for j in range(8):
                uu = ub[e, pl.ds(16 * j, 16)]
                vv = vb[e, pl.ds(16 * j, 16)]
                m = (uu * vv) * wv[j]
                acc = m if j == 0 else acc + m
                acc = acc + _perm(m, hi_idx)
            mm = acc + _perm(acc, p4_idx)
            nn = mm + _perm(mm, p2_idx)
            res = nn + _perm(nn, p1_idx)
            plsc.store_scatter(
                ob, [jnp.zeros((L,), jnp.int32) + (e + ci * CHUNK)], res,
                mask=lane0)

        pend = nxt
    pltpu.sync_copy(ob, sums_hbm.at[pl.ds(wid * E_PER_W, E_PER_W)])


@jax.jit
def _edge_sums(heads, tails, in_embed, out_embed, diag_w):
    mesh = plsc.VectorSubcoreMesh(core_axis_name="c", subcore_axis_name="s")
    k = functools.partial(
        pl.kernel,
        mesh=mesh,
        compiler_params=pltpu.CompilerParams(needs_layout_passes=False,
                                             use_tc_tiling_on_sc=True),
        out_type=jax.ShapeDtypeStruct((B,), jnp.float32),
        scratch_types=[
            pltpu.VMEM((CHUNK,), jnp.int32),
            pltpu.VMEM((CHUNK,), jnp.int32),
            pltpu.VMEM((CHUNK, D), jnp.float32),
            pltpu.VMEM((CHUNK, D), jnp.float32),
            pltpu.VMEM((CHUNK,), jnp.int32),
            pltpu.VMEM((CHUNK,), jnp.int32),
            pltpu.VMEM((CHUNK, D), jnp.float32),
            pltpu.VMEM((CHUNK, D), jnp.float32),
            pltpu.VMEM((D,), jnp.float32),
            pltpu.VMEM((E_PER_W,), jnp.float32),
            pltpu.SemaphoreType.DMA,
            pltpu.SemaphoreType.DMA,
            pltpu.SemaphoreType.DMA,
            pltpu.SemaphoreType.DMA,
        ],
    )(_sums_kernel)
    return k(heads, tails, in_embed, out_embed, diag_w)


ST = 16
SE = B // ST
NV = SE // 16
RB = 8192
SLICE = RB // ST
SCAN_BASE = 1


def _sort_kernel(keys_hbm, order_hbm,
                 keyb, valb, posb, hist, hblk, p2b, totb, stot, tsr, vtmp,
                 prow, tot,
                 G, P2g, STot2, TS2, keyS, valS):
    cid = lax.axis_index("c")
    sid = lax.axis_index("s")
    lane = lax.iota(jnp.int32, L)

    @pl.when(cid == 0)
    def _body():
        t = sid

        def one_pass(shift, first, last):
            scope = jax.named_scope
            ctxA = scope(f"phA_{shift}"); ctxA.__enter__()
            @plsc.parallel_loop(0, RB // 16, unroll=8)
            def zbody(i):
                hist[pl.ds(16 * i, 16)] = jnp.zeros((16,), jnp.int32)

            if first:
                pltpu.sync_copy(keys_hbm.at[pl.ds(t * SE, SE)], keyb)

                @plsc.parallel_loop(0, NV, unroll=4)
                def vbody(j):
                    valb[pl.ds(16 * j, 16)] = t * SE + 16 * j + lane
            else:
                pltpu.sync_copy(keyS.at[pl.ds(t * SE, SE)], keyb)
                pltpu.sync_copy(valS.at[pl.ds(t * SE, SE)], valb)

            def hbody(j, c):
                kv = keyb[pl.ds(16 * j, 16)]
                dv = lax.shift_right_logical(kv, shift) & (RB - 1)
                rank, lastm = plsc.scan_count(dv)
                cur = plsc.load_gather(hist, [dv])
                plsc.store_scatter(hist, [dv],
                                   cur + rank + (1 - SCAN_BASE), mask=lastm)
                return c
            lax.fori_loop(0, NV, hbody, 0, unroll=4)
            pltpu.sync_copy(hist, G.at[t])
            ctxA.__exit__(None, None, None)
            plsc.subcore_barrier()

            ctxB = scope(f"phB_{shift}"); ctxB.__enter__()
            pltpu.sync_copy(G.at[:, pl.ds(SLICE * t, SLICE)], hblk)

            @plsc.parallel_loop(0, SLICE // 16, unroll=2)
            def pbody(j):
                acc = jnp.zeros((16,), jnp.int32)
                for t2 in range(ST):
                    v = hblk[t2, pl.ds(16 * j, 16)]
                    p2b[t2, pl.ds(16 * j, 16)] = acc
                    acc = acc + v
                totb[pl.ds(16 * j, 16)] = acc
            pltpu.sync_copy(p2b, P2g.at[:, pl.ds(SLICE * t, SLICE)])

            def sbody(i, c):
                v = totb[pl.ds(16 * i, 16)]
                incl = jnp.cumsum(v)
                stot[pl.ds(16 * i, 16)] = incl - v + c
                return c + jnp.sum(v)
            gt = lax.fori_loop(0, SLICE // 16, sbody, jnp.int32(0),
                               unroll=False)
            pltpu.sync_copy(stot, STot2.at[pl.ds(SLICE * t, SLICE)])
            vtmp[...] = jnp.zeros((16,), jnp.int32) + gt
            pltpu.sync_copy(vtmp.at[pl.ds(0, 8)], TS2.at[pl.ds(8 * t, 8)])
            ctxB.__exit__(None, None, None)
            plsc.subcore_barrier()

            ctxC = scope(f"phC_{shift}"); ctxC.__enter__()
            pltpu.sync_copy(STot2, tot)
            pltpu.sync_copy(P2g.at[t], prow)
            pltpu.sync_copy(TS2, tsr)
            tsv = plsc.load_gather(tsr, [lane * 8])
            vtmp[...] = jnp.cumsum(tsv) - tsv

            @plsc.parallel_loop(0, RB // 16, unroll=8)
            def obody(j):
                sb = plsc.load_gather(
                    vtmp, [jnp.zeros((16,), jnp.int32) + (j // 32)])
                hist[pl.ds(16 * j, 16)] = (tot[pl.ds(16 * j, 16)]
                                           + prow[pl.ds(16 * j, 16)] + sb)

            def rbody(j, c):
                kv = keyb[pl.ds(16 * j, 16)]
                dv = lax.shift_right_logical(kv, shift) & (RB - 1)
                rank, lastm = plsc.scan_count(dv)
                base = plsc.load_gather(hist, [dv])
                plsc.store_scatter(hist, [dv],
                                   base + rank + (1 - SCAN_BASE), mask=lastm)
                posb[pl.ds(16 * j, 16)] = base + rank - SCAN_BASE
                return c
            lax.fori_loop(0, NV, rbody, 0, unroll=4)

            ctxC.__exit__(None, None, None)
            ctxD = scope(f"phD_{shift}"); ctxD.__enter__()
            if last:
                pltpu.sync_copy(valb, valS.at[posb])
                plsc.subcore_barrier()
                pltpu.sync_copy(valS.at[pl.ds(t * SE, SE)], posb)
                pltpu.sync_copy(posb, order_hbm.at[pl.ds(t * SE, SE)])
            else:
                pltpu.sync_copy(keyb, keyS.at[posb])
                pltpu.sync_copy(valb, valS.at[posb])
                plsc.subcore_barrier()
            ctxD.__exit__(None, None, None)

        one_pass(0, True, False)
        one_pass(13, False, True)


@jax.jit
def _sc_sort(keys):
    mesh = plsc.VectorSubcoreMesh(core_axis_name="c", subcore_axis_name="s")
    k = functools.partial(
        pl.kernel,
        mesh=mesh,
        compiler_params=pltpu.CompilerParams(needs_layout_passes=False),
        out_type=jax.ShapeDtypeStruct((B,), jnp.int32),
        scratch_types=[
            pltpu.VMEM((SE,), jnp.int32),
            pltpu.VMEM((SE,), jnp.int32),
            pltpu.VMEM((SE,), jnp.int32),
            pltpu.VMEM((RB,), jnp.int32),
            pltpu.VMEM((ST, SLICE), jnp.int32),
            pltpu.VMEM((ST, SLICE), jnp.int32),
            pltpu.VMEM((SLICE,), jnp.int32),
            pltpu.VMEM((SLICE,), jnp.int32),
            pltpu.VMEM((8 * ST,), jnp.int32),
            pltpu.VMEM((16,), jnp.int32),
            pltpu.VMEM((RB,), jnp.int32),
            pltpu.VMEM((RB,), jnp.int32),
            pltpu.VMEM_SHARED((ST, RB), jnp.int32),
            pltpu.VMEM_SHARED((ST, RB), jnp.int32),
            pltpu.VMEM_SHARED((RB,), jnp.int32),
            pltpu.VMEM_SHARED((8 * ST,), jnp.int32),
            pltpu.VMEM_SHARED((B,), jnp.int32),
            pltpu.VMEM_SHARED((B,), jnp.int32),
        ],
    )(_sort_kernel)
    return k(keys)


def kernel(heads, tails, in_embed, out_embed, diag_w):
    sums = _edge_sums(heads, tails, in_embed, out_embed, diag_w)
    log_target = jax.nn.sigmoid(sums)
    keys = jnp.int32(0x3F7FFFFF) - lax.bitcast_convert_type(
        log_target, jnp.int32)
    order = _sc_sort(keys)
    return log_target, order


# speedup vs baseline: 1.0001x; 1.0001x over previous
import functools

import jax
import jax.numpy as jnp
from jax import lax
from jax.experimental import pallas as pl
from jax.experimental.pallas import tpu as pltpu
from jax.experimental.pallas import tpu_sc as plsc

D = 128
B = 16384

_info = plsc.get_sparse_core_info()
NC, NS, L = _info.num_cores, _info.num_subcores, _info.num_lanes
NW = NC * NS
E_PER_W = B // NW
CHUNK = 128
N_CHUNKS = E_PER_W // CHUNK

_IB = lax.GatherScatterMode.PROMISE_IN_BOUNDS


_DNUMS = lax.GatherDimensionNumbers(
    offset_dims=(), collapsed_slice_dims=(0,), start_index_map=(0,))


def _perm(x, idx):
    return lax.gather(x, idx[:, None], _DNUMS, slice_sizes=(1,), mode=_IB)


def _sums_kernel(heads_hbm, tails_hbm, in_hbm, out_hbm, w_hbm, sums_hbm,
                 hv0, tv0, ub0, vb0, hv1, tv1, ub1, vb1, wb, ob,
                 su0, sv0, su1, sv1):
    wid = lax.axis_index("s") * NC + lax.axis_index("c")
    lane = lax.iota(jnp.int32, L)
    lane0 = lane == 0
    hi_idx = (lane & 7) + 8
    p4_idx = (lane & 3) + 4
    p2_idx = (lane & 1) + 2
    p1_idx = (lane & 0) + 1

    pltpu.sync_copy(w_hbm, wb)
    wv = [wb[pl.ds(16 * j, 16)] for j in range(8)]
    bufs = [(hv0, tv0, ub0, vb0, su0, sv0), (hv1, tv1, ub1, vb1, su1, sv1)]

    def start(ci):
        hv, tv, ub, vb, su, sv = bufs[ci & 1]
        base = wid * E_PER_W + ci * CHUNK
        pltpu.sync_copy(heads_hbm.at[pl.ds(base, CHUNK)], hv)
        pltpu.sync_copy(tails_hbm.at[pl.ds(base, CHUNK)], tv)
        return (pltpu.async_copy(in_hbm.at[hv], ub, su),
                pltpu.async_copy(out_hbm.at[tv], vb, sv))

    pend = start(0)
    for ci in range(N_CHUNKS):
        nxt = start(ci + 1) if ci + 1 < N_CHUNKS else None
        pend[0].wait()
        pend[1].wait()
        _, _, ub, vb, _, _ = bufs[ci & 1]

        @plsc.parallel_loop(0, CHUNK, unroll=4)
        def edge_body(e):
            acc = None
            for j in range(8):
                uu = ub[e, pl.ds(16 * j, 16)]
                vv = vb[e, pl.ds(16 * j, 16)]
                m = (uu * vv) * wv[j]
                acc = m if j == 0 else acc + m
                acc = acc + _perm(m, hi_idx)
            mm = acc + _perm(acc, p4_idx)
            nn = mm + _perm(mm, p2_idx)
            res = nn + _perm(nn, p1_idx)
            plsc.store_scatter(
                ob, [jnp.zeros((L,), jnp.int32) + (e + ci * CHUNK)], res,
                mask=lane0)

        pend = nxt
    pltpu.sync_copy(ob, sums_hbm.at[pl.ds(wid * E_PER_W, E_PER_W)])


@jax.jit
def _edge_sums(heads, tails, in_embed, out_embed, diag_w):
    mesh = plsc.VectorSubcoreMesh(core_axis_name="c", subcore_axis_name="s")
    k = functools.partial(
        pl.kernel,
        mesh=mesh,
        compiler_params=pltpu.CompilerParams(needs_layout_passes=False),
        out_type=jax.ShapeDtypeStruct((B,), jnp.float32),
        scratch_types=[
            pltpu.VMEM((CHUNK,), jnp.int32),
            pltpu.VMEM((CHUNK,), jnp.int32),
            pltpu.VMEM((CHUNK, D), jnp.float32),
            pltpu.VMEM((CHUNK, D), jnp.float32),
            pltpu.VMEM((CHUNK,), jnp.int32),
            pltpu.VMEM((CHUNK,), jnp.int32),
            pltpu.VMEM((CHUNK, D), jnp.float32),
            pltpu.VMEM((CHUNK, D), jnp.float32),
            pltpu.VMEM((D,), jnp.float32),
            pltpu.VMEM((E_PER_W,), jnp.float32),
            pltpu.SemaphoreType.DMA,
            pltpu.SemaphoreType.DMA,
            pltpu.SemaphoreType.DMA,
            pltpu.SemaphoreType.DMA,
        ],
    )(_sums_kernel)
    return k(heads, tails, in_embed, out_embed, diag_w)


ST = 16
SE = B // ST
NV = SE // 16
RB = 8192
SLICE = RB // ST
SCAN_BASE = 1


def _sort_kernel(keys_hbm, order_hbm,
                 keyb, valb, posb, hist, hblk, p2b, totb, stot, tsr, vtmp,
                 prow, tot,
                 G, P2g, STot2, TS2, keyS, valS):
    cid = lax.axis_index("c")
    sid = lax.axis_index("s")
    lane = lax.iota(jnp.int32, L)

    @pl.when(cid == 0)
    def _body():
        t = sid

        def one_pass(shift, first, last):
            scope = jax.named_scope
            ctxA = scope(f"phA_{shift}"); ctxA.__enter__()
            @plsc.parallel_loop(0, RB // 16, unroll=8)
            def zbody(i):
                hist[pl.ds(16 * i, 16)] = jnp.zeros((16,), jnp.int32)

            if first:
                pltpu.sync_copy(keys_hbm.at[pl.ds(t * SE, SE)], keyb)

                @plsc.parallel_loop(0, NV, unroll=4)
                def vbody(j):
                    valb[pl.ds(16 * j, 16)] = t * SE + 16 * j + lane
            else:
                pltpu.sync_copy(keyS.at[pl.ds(t * SE, SE)], keyb)
                pltpu.sync_copy(valS.at[pl.ds(t * SE, SE)], valb)

            def hbody(j, c):
                kv = keyb[pl.ds(16 * j, 16)]
                dv = lax.shift_right_logical(kv, shift) & (RB - 1)
                rank, lastm = plsc.scan_count(dv)
                cur = plsc.load_gather(hist, [dv])
                plsc.store_scatter(hist, [dv],
                                   cur + rank + (1 - SCAN_BASE), mask=lastm)
                return c
            lax.fori_loop(0, NV, hbody, 0, unroll=4)
            pltpu.sync_copy(hist, G.at[t])
            ctxA.__exit__(None, None, None)
            plsc.subcore_barrier()

            ctxB = scope(f"phB_{shift}"); ctxB.__enter__()
            pltpu.sync_copy(G.at[:, pl.ds(SLICE * t, SLICE)], hblk)

            @plsc.parallel_loop(0, SLICE // 16, unroll=2)
            def pbody(j):
                acc = jnp.zeros((16,), jnp.int32)
                for t2 in range(ST):
                    v = hblk[t2, pl.ds(16 * j, 16)]
                    p2b[t2, pl.ds(16 * j, 16)] = acc
                    acc = acc + v
                totb[pl.ds(16 * j, 16)] = acc
            pltpu.sync_copy(p2b, P2g.at[:, pl.ds(SLICE * t, SLICE)])

            def sbody(i, c):
                v = totb[pl.ds(16 * i, 16)]
                incl = jnp.cumsum(v)
                stot[pl.ds(16 * i, 16)] = incl - v + c
                return c + jnp.sum(v)
            gt = lax.fori_loop(0, SLICE // 16, sbody, jnp.int32(0),
                               unroll=False)
            pltpu.sync_copy(stot, STot2.at[pl.ds(SLICE * t, SLICE)])
            vtmp[...] = jnp.zeros((16,), jnp.int32) + gt
            pltpu.sync_copy(vtmp.at[pl.ds(0, 8)], TS2.at[pl.ds(8 * t, 8)])
            ctxB.__exit__(None, None, None)
            plsc.subcore_barrier()

            ctxC = scope(f"phC_{shift}"); ctxC.__enter__()
            pltpu.sync_copy(STot2, tot)
            pltpu.sync_copy(P2g.at[t], prow)
            pltpu.sync_copy(TS2, tsr)
            tsv = plsc.load_gather(tsr, [lane * 8])
            vtmp[...] = jnp.cumsum(tsv) - tsv

            @plsc.parallel_loop(0, RB // 16, unroll=8)
            def obody(j):
                sb = plsc.load_gather(
                    vtmp, [jnp.zeros((16,), jnp.int32) + (j // 32)])
                hist[pl.ds(16 * j, 16)] = (tot[pl.ds(16 * j, 16)]
                                           + prow[pl.ds(16 * j, 16)] + sb)

            def rbody(j, c):
                kv = keyb[pl.ds(16 * j, 16)]
                dv = lax.shift_right_logical(kv, shift) & (RB - 1)
                rank, lastm = plsc.scan_count(dv)
                base = plsc.load_gather(hist, [dv])
                plsc.store_scatter(hist, [dv],
                                   base + rank + (1 - SCAN_BASE), mask=lastm)
                posb[pl.ds(16 * j, 16)] = base + rank - SCAN_BASE
                return c
            lax.fori_loop(0, NV, rbody, 0, unroll=4)

            ctxC.__exit__(None, None, None)
            ctxD = scope(f"phD_{shift}"); ctxD.__enter__()
            if last:
                pltpu.sync_copy(valb, valS.at[posb])
                plsc.subcore_barrier()
                pltpu.sync_copy(valS.at[pl.ds(t * SE, SE)], posb)
                pltpu.sync_copy(posb, order_hbm.at[pl.ds(t * SE, SE)])
            else:
                pltpu.sync_copy(keyb, keyS.at[posb])
                pltpu.sync_copy(valb, valS.at[posb])
                plsc.subcore_barrier()
            ctxD.__exit__(None, None, None)

        one_pass(0, True, False)
        one_pass(13, False, True)


@jax.jit
def _sc_sort(keys):
    mesh = plsc.VectorSubcoreMesh(core_axis_name="c", subcore_axis_name="s")
    k = functools.partial(
        pl.kernel,
        mesh=mesh,
        compiler_params=pltpu.CompilerParams(needs_layout_passes=False),
        out_type=jax.ShapeDtypeStruct((B,), jnp.int32),
        scratch_types=[
            pltpu.VMEM((SE,), jnp.int32),
            pltpu.VMEM((SE,), jnp.int32),
            pltpu.VMEM((SE,), jnp.int32),
            pltpu.VMEM((RB,), jnp.int32),
            pltpu.VMEM((ST, SLICE), jnp.int32),
            pltpu.VMEM((ST, SLICE), jnp.int32),
            pltpu.VMEM((SLICE,), jnp.int32),
            pltpu.VMEM((SLICE,), jnp.int32),
            pltpu.VMEM((8 * ST,), jnp.int32),
            pltpu.VMEM((16,), jnp.int32),
            pltpu.VMEM((RB,), jnp.int32),
            pltpu.VMEM((RB,), jnp.int32),
            pltpu.VMEM_SHARED((ST, RB), jnp.int32),
            pltpu.VMEM_SHARED((ST, RB), jnp.int32),
            pltpu.VMEM_SHARED((RB,), jnp.int32),
            pltpu.VMEM_SHARED((8 * ST,), jnp.int32),
            pltpu.VMEM_SHARED((B,), jnp.int32),
            pltpu.VMEM_SHARED((B,), jnp.int32),
        ],
    )(_sort_kernel)
    return k(keys)


def kernel(heads, tails, in_embed, out_embed, diag_w):
    sums = _edge_sums(heads, tails, in_embed, out_embed, diag_w)
    log_target = jax.nn.sigmoid(sums)
    keys = jnp.int32(0x3F7FFFFF) - lax.bitcast_convert_type(
        log_target, jnp.int32)
    order = _sc_sort(keys)
    return log_target, order


# speedup vs baseline: 1.0025x; 1.0024x over previous
import functools

import jax
import jax.numpy as jnp
from jax import lax
from jax.experimental import pallas as pl
from jax.experimental.pallas import tpu as pltpu
from jax.experimental.pallas import tpu_sc as plsc

D = 128
B = 16384

_info = plsc.get_sparse_core_info()
NC, NS, L = _info.num_cores, _info.num_subcores, _info.num_lanes
NW = NC * NS
E_PER_W = B // NW
CHUNK = 128
N_CHUNKS = E_PER_W // CHUNK

_IB = lax.GatherScatterMode.PROMISE_IN_BOUNDS


_DNUMS = lax.GatherDimensionNumbers(
    offset_dims=(), collapsed_slice_dims=(0,), start_index_map=(0,))


def _perm(x, idx):
    return lax.gather(x, idx[:, None], _DNUMS, slice_sizes=(1,), mode=_IB)


def _sums_kernel(heads_hbm, tails_hbm, in_hbm, out_hbm, w_hbm, sums_hbm,
                 hv0, tv0, ub0, vb0, hv1, tv1, ub1, vb1, wb, ob,
                 su0, sv0, su1, sv1):
    wid = lax.axis_index("s") * NC + lax.axis_index("c")
    lane = lax.iota(jnp.int32, L)
    lane0 = lane == 0
    hi_idx = (lane & 7) + 8
    p4_idx = (lane & 3) + 4
    p2_idx = (lane & 1) + 2
    p1_idx = (lane & 0) + 1

    pltpu.sync_copy(w_hbm, wb)
    wv = [wb[pl.ds(16 * j, 16)] for j in range(8)]
    bufs = [(hv0, tv0, ub0, vb0, su0, sv0), (hv1, tv1, ub1, vb1, su1, sv1)]

    def start(ci):
        hv, tv, ub, vb, su, sv = bufs[ci & 1]
        base = wid * E_PER_W + ci * CHUNK
        pltpu.sync_copy(heads_hbm.at[pl.ds(base, CHUNK)], hv)
        pltpu.sync_copy(tails_hbm.at[pl.ds(base, CHUNK)], tv)
        return (pltpu.async_copy(in_hbm.at[hv], ub, su),
                pltpu.async_copy(out_hbm.at[tv], vb, sv))

    pend = start(0)
    for ci in range(N_CHUNKS):
        nxt = start(ci + 1) if ci + 1 < N_CHUNKS else None
        pend[0].wait()
        pend[1].wait()
        _, _, ub, vb, _, _ = bufs[ci & 1]

        @plsc.parallel_loop(0, CHUNK, unroll=4)
        def edge_body(e):
            acc = None
            for j in range(8):
                uu = ub[e, pl.ds(16 * j, 16)]
                vv = vb[e, pl.ds(16 * j, 16)]
                m = (uu * vv) * wv[j]
                acc = m if j == 0 else acc + m
                acc = acc + _perm(m, hi_idx)
            mm = acc + _perm(acc, p4_idx)
            nn = mm + _perm(mm, p2_idx)
            res = nn + _perm(nn, p1_idx)
            plsc.store_scatter(
                ob, [jnp.zeros((L,), jnp.int32) + (e + ci * CHUNK)], res,
                mask=lane0)

        pend = nxt
    pltpu.sync_copy(ob, sums_hbm.at[pl.ds(wid * E_PER_W, E_PER_W)])


@jax.jit
def _edge_sums(heads, tails, in_embed, out_embed, diag_w):
    mesh = plsc.VectorSubcoreMesh(core_axis_name="c", subcore_axis_name="s")
    k = functools.partial(
        pl.kernel,
        mesh=mesh,
        compiler_params=pltpu.CompilerParams(needs_layout_passes=False),
        out_type=jax.ShapeDtypeStruct((B,), jnp.float32),
        scratch_types=[
            pltpu.VMEM((CHUNK,), jnp.int32),
            pltpu.VMEM((CHUNK,), jnp.int32),
            pltpu.VMEM((CHUNK, D), jnp.float32),
            pltpu.VMEM((CHUNK, D), jnp.float32),
            pltpu.VMEM((CHUNK,), jnp.int32),
            pltpu.VMEM((CHUNK,), jnp.int32),
            pltpu.VMEM((CHUNK, D), jnp.float32),
            pltpu.VMEM((CHUNK, D), jnp.float32),
            pltpu.VMEM((D,), jnp.float32),
            pltpu.VMEM((E_PER_W,), jnp.float32),
            pltpu.SemaphoreType.DMA,
            pltpu.SemaphoreType.DMA,
            pltpu.SemaphoreType.DMA,
            pltpu.SemaphoreType.DMA,
        ],
    )(_sums_kernel)
    return k(heads, tails, in_embed, out_embed, diag_w)


ST = 16
SE = B // ST
NV = SE // 16
RB = 8192
SLICE = RB // ST
SCAN_BASE = 1


def _sort_kernel(keys_hbm, order_hbm,
                 keyb, valb, posb, hist, hblk, p2b, totb, stot, tsr, vtmp,
                 prow, tot,
                 G, P2g, STot2, TS2, keyS, valS):
    cid = lax.axis_index("c")
    sid = lax.axis_index("s")
    lane = lax.iota(jnp.int32, L)

    @pl.when(cid == 0)
    def _body():
        t = sid

        def one_pass(shift, first, last):
            @plsc.parallel_loop(0, RB // 16, unroll=8)
            def zbody(i):
                hist[pl.ds(16 * i, 16)] = jnp.zeros((16,), jnp.int32)

            if first:
                pltpu.sync_copy(keys_hbm.at[pl.ds(t * SE, SE)], keyb)

                @plsc.parallel_loop(0, NV, unroll=4)
                def vbody(j):
                    valb[pl.ds(16 * j, 16)] = t * SE + 16 * j + lane
            else:
                pltpu.sync_copy(keyS.at[pl.ds(t * SE, SE)], keyb)
                pltpu.sync_copy(valS.at[pl.ds(t * SE, SE)], valb)

            def hbody(j, c):
                kv = keyb[pl.ds(16 * j, 16)]
                dv = lax.shift_right_logical(kv, shift) & (RB - 1)
                rank, lastm = plsc.scan_count(dv)
                cur = plsc.load_gather(hist, [dv])
                plsc.store_scatter(hist, [dv],
                                   cur + rank + (1 - SCAN_BASE), mask=lastm)
                return c
            lax.fori_loop(0, NV, hbody, 0, unroll=4)
            pltpu.sync_copy(hist, G.at[t])
            plsc.subcore_barrier()

            pltpu.sync_copy(G.at[:, pl.ds(SLICE * t, SLICE)], hblk)

            @plsc.parallel_loop(0, SLICE // 16, unroll=2)
            def pbody(j):
                acc = jnp.zeros((16,), jnp.int32)
                for t2 in range(ST):
                    v = hblk[t2, pl.ds(16 * j, 16)]
                    p2b[t2, pl.ds(16 * j, 16)] = acc
                    acc = acc + v
                totb[pl.ds(16 * j, 16)] = acc
            pltpu.sync_copy(p2b, P2g.at[:, pl.ds(SLICE * t, SLICE)])

            def sbody(i, c):
                v = totb[pl.ds(16 * i, 16)]
                incl = jnp.cumsum(v)
                stot[pl.ds(16 * i, 16)] = incl - v + c
                return c + jnp.sum(v)
            gt = lax.fori_loop(0, SLICE // 16, sbody, jnp.int32(0),
                               unroll=False)
            pltpu.sync_copy(stot, STot2.at[pl.ds(SLICE * t, SLICE)])
            vtmp[...] = jnp.zeros((16,), jnp.int32) + gt
            pltpu.sync_copy(vtmp.at[pl.ds(0, 8)], TS2.at[pl.ds(8 * t, 8)])
            plsc.subcore_barrier()

            pltpu.sync_copy(STot2, tot)
            pltpu.sync_copy(P2g.at[t], prow)
            pltpu.sync_copy(TS2, tsr)
            tsv = plsc.load_gather(tsr, [lane * 8])
            vtmp[...] = jnp.cumsum(tsv) - tsv

            @plsc.parallel_loop(0, RB // 16, unroll=8)
            def obody(j):
                sb = plsc.load_gather(
                    vtmp, [jnp.zeros((16,), jnp.int32) + (j // 32)])
                hist[pl.ds(16 * j, 16)] = (tot[pl.ds(16 * j, 16)]
                                           + prow[pl.ds(16 * j, 16)] + sb)

            def rbody(j, c):
                kv = keyb[pl.ds(16 * j, 16)]
                dv = lax.shift_right_logical(kv, shift) & (RB - 1)
                rank, lastm = plsc.scan_count(dv)
                base = plsc.load_gather(hist, [dv])
                plsc.store_scatter(hist, [dv],
                                   base + rank + (1 - SCAN_BASE), mask=lastm)
                posb[pl.ds(16 * j, 16)] = base + rank - SCAN_BASE
                return c
            lax.fori_loop(0, NV, rbody, 0, unroll=4)

            if last:
                pltpu.sync_copy(valb, valS.at[posb])
                plsc.subcore_barrier()
                pltpu.sync_copy(valS.at[pl.ds(t * SE, SE)], posb)
                pltpu.sync_copy(posb, order_hbm.at[pl.ds(t * SE, SE)])
            else:
                pltpu.sync_copy(keyb, keyS.at[posb])
                pltpu.sync_copy(valb, valS.at[posb])
                plsc.subcore_barrier()

        one_pass(0, True, False)
        one_pass(13, False, True)


@jax.jit
def _sc_sort(keys):
    mesh = plsc.VectorSubcoreMesh(core_axis_name="c", subcore_axis_name="s")
    k = functools.partial(
        pl.kernel,
        mesh=mesh,
        compiler_params=pltpu.CompilerParams(needs_layout_passes=False),
        out_type=jax.ShapeDtypeStruct((B,), jnp.int32),
        scratch_types=[
            pltpu.VMEM((SE,), jnp.int32),
            pltpu.VMEM((SE,), jnp.int32),
            pltpu.VMEM((SE,), jnp.int32),
            pltpu.VMEM((RB,), jnp.int32),
            pltpu.VMEM((ST, SLICE), jnp.int32),
            pltpu.VMEM((ST, SLICE), jnp.int32),
            pltpu.VMEM((SLICE,), jnp.int32),
            pltpu.VMEM((SLICE,), jnp.int32),
            pltpu.VMEM((8 * ST,), jnp.int32),
            pltpu.VMEM((16,), jnp.int32),
            pltpu.VMEM((RB,), jnp.int32),
            pltpu.VMEM((RB,), jnp.int32),
            pltpu.VMEM_SHARED((ST, RB), jnp.int32),
            pltpu.VMEM_SHARED((ST, RB), jnp.int32),
            pltpu.VMEM_SHARED((RB,), jnp.int32),
            pltpu.VMEM_SHARED((8 * ST,), jnp.int32),
            pltpu.VMEM_SHARED((B,), jnp.int32),
            pltpu.VMEM_SHARED((B,), jnp.int32),
        ],
    )(_sort_kernel)
    return k(keys)


def kernel(heads, tails, in_embed, out_embed, diag_w):
    sums = _edge_sums(heads, tails, in_embed, out_embed, diag_w)
    log_target = jax.nn.sigmoid(sums)
    keys = jnp.int32(0x3F7FFFFF) - lax.bitcast_convert_type(
        log_target, jnp.int32)
    order = _sc_sort(keys)
    return log_target, order
